# baseline (speedup 1.0000x reference)
import math
import functools

import jax
import jax.numpy as jnp
from jax.experimental import pallas as pl
from jax.experimental.pallas import tpu as pltpu

_B, _N, _C = 16, 1024, 3
_HID = 64
_H = 4
_HS = _HID // _H
_NUM_BLOCKS = 2
_STEPS = 8
_INV_SQRT_HS = 1.0 / math.sqrt(_HS)
_LOG2E = math.log2(math.e)
_LOGZ = -0.5 * math.log(2.0 * math.pi)


def _mlp_t(ws, zt, g1):
    (w1, b1), (w2, b2), (w3, b3) = ws
    t1 = jnp.tanh(jnp.dot(w1, zt, preferred_element_type=jnp.float32) + b1)
    u1 = g1 * (1.0 - t1 * t1)
    t2 = jnp.tanh(jnp.dot(w2, t1, preferred_element_type=jnp.float32) + b2)
    u2 = jnp.dot(w2, u1, preferred_element_type=jnp.float32) * (1.0 - t2 * t2)
    out = jnp.dot(w3, t2, preferred_element_type=jnp.float32) + b3
    dout = jnp.dot(w3, u2, preferred_element_type=jnp.float32)
    return out, dout


def _dotg(a, b, dims):
    return jax.lax.dot_general(a, b, (dims, ((), ())),
                               preferred_element_type=jnp.float32)


def _cnf_kernel(xt_ref, et_ref, *rest):
    out_ref = rest[-1]
    wrefs = rest[:-1]

    zt = xt_ref[0]
    logp = jnp.float32(0.0)

    per_blk = 21
    for i in range(_NUM_BLOCKS):
        r = wrefs[i * per_blk:(i + 1) * per_blk]
        kw = [(r[0][...], r[1][...]), (r[2][...], r[3][...]), (r[4][...], r[5][...])]
        qw = [(r[6][...], r[7][...]), (r[8][...], r[9][...]), (r[10][...], r[11][...])]
        vw = [(r[12][...], r[13][...]), (r[14][...], r[15][...]), (r[16][...], r[17][...])]
        wm, bm = r[18][...], r[19][...]
        sqrt_t = r[20][0, 0]
        dt = (sqrt_t * sqrt_t) / (_STEPS - 1)
        et = et_ref[i, 0]

        ones_row = jnp.ones((1, _N), dtype=jnp.float32)
        gk = jnp.dot(kw[0][0], et, preferred_element_type=jnp.float32)
        gq = jnp.dot(qw[0][0], et, preferred_element_type=jnp.float32)
        gv = jnp.dot(vw[0][0], et, preferred_element_type=jnp.float32)

        def step(_, carry, kw=kw, qw=qw, vw=vw, wm=wm, bm=bm, et=et,
                 ones_row=ones_row, gk=gk, gq=gq, gv=gv):
            zt, logp = carry
            kt, dkt = _mlp_t(kw, zt, gk)
            qt, dqt = _mlp_t(qw, zt, gq)
            vt, dvt = _mlp_t(vw, zt, gv)

            rt_heads = []
            drt_heads = []
            for h in range(_H):
                sl = slice(h * _HS, (h + 1) * _HS)
                qh = qt[sl] * (_INV_SQRT_HS * _LOG2E)
                kh = kt[sl]
                kh4 = kt[sl] * _INV_SQRT_HS
                dkh4 = dkt[sl] * _INV_SQRT_HS
                vh = vt[sl]
                dvh = dvt[sl]

                st = _dotg(kh, qh, ((0,), (0,)))
                m = jnp.max(st, axis=0, keepdims=True)
                p = jnp.exp2(st - m).astype(jnp.bfloat16)

                dst = _dotg(jnp.concatenate([kh4, dkh4], axis=0),
                            jnp.concatenate([dqt[sl], qt[sl]], axis=0),
                            ((0,), (0,))).astype(jnp.bfloat16)
                u = p * dst

                av3 = _dotg(
                    jnp.concatenate([vh, dvh, ones_row],
                                    axis=0).astype(jnp.bfloat16), p,
                    ((1,), (0,)))
                rd = 1.0 / av3[2 * _HS:]
                attn_t = av3[:_HS] * rd
                a_dv = av3[_HS:2 * _HS] * rd

                vu2 = _dotg(
                    jnp.concatenate([vh, ones_row],
                                    axis=0).astype(jnp.bfloat16), u,
                    ((1,), (0,)))
                dav = (vu2[:_HS] - attn_t * vu2[_HS:]) * rd

                rt_heads.append(qt[sl] + attn_t)
                drt_heads.append(dqt[sl] + a_dv + dav)

            rt = jnp.concatenate(rt_heads, axis=0)
            drt = jnp.concatenate(drt_heads, axis=0)
            tt = jnp.tanh(rt)
            dx = jnp.dot(wm, tt, preferred_element_type=jnp.float32) + bm
            ddx = jnp.dot(wm, drt * (1.0 - tt * tt),
                          preferred_element_type=jnp.float32)
            div = jnp.sum(ddx * et)
            return zt + dt * dx, logp - dt * div

        zt, logp = jax.lax.fori_loop(0, _STEPS - 1, step, (zt, logp))

    logpz = jnp.sum(_LOGZ - 0.5 * zt * zt)
    out_ref[0] = jnp.reshape(logpz - logp, (1, 1))


def kernel(x, e, params):
    xt = x.transpose(0, 2, 1)
    et = e.transpose(0, 1, 3, 2)

    flat = []
    for blk in params:
        for name in ("K", "Q", "V"):
            for (w, b) in blk[name]:
                flat.append(w.T)
                flat.append(b.reshape(-1, 1))
        wm, bm = blk["M"][0]
        flat.append(wm.T)
        flat.append(bm.reshape(-1, 1))
        flat.append(blk["sqrt_T"].reshape(1, 1))

    n = _N
    n_cores = 2
    per_core = _B // n_cores
    in_specs = [
        pl.BlockSpec((1, _C, n), lambda c, j: (c * per_core + j, 0, 0)),
        pl.BlockSpec((_NUM_BLOCKS, 1, _C, n),
                     lambda c, j: (0, c * per_core + j, 0, 0)),
    ]
    for a in flat:
        in_specs.append(
            pl.BlockSpec(a.shape,
                         functools.partial(lambda nd, c, j: (0,) * nd, a.ndim)))

    out = pl.pallas_call(
        _cnf_kernel,
        grid=(n_cores, per_core),
        in_specs=in_specs,
        out_specs=pl.BlockSpec((1, 1, 1), lambda c, j: (c * per_core + j, 0, 0)),
        out_shape=jax.ShapeDtypeStruct((_B, 1, 1), jnp.float32),
        compiler_params=pltpu.CompilerParams(
            dimension_semantics=("parallel", "arbitrary"),
            vmem_limit_bytes=56 * 1024 * 1024,
        ),
    )(xt, et, *flat)
    return out.reshape(_B)


# speedup vs baseline: 1.1076x; 1.1076x over previous
import math
import functools

import jax
import jax.numpy as jnp
from jax.experimental import pallas as pl
from jax.experimental.pallas import tpu as pltpu

_B, _N, _C = 16, 1024, 3
_HID = 64
_H = 4
_HS = _HID // _H
_NUM_BLOCKS = 2
_STEPS = 8
_GB = 2
_INV_SQRT_HS = 1.0 / math.sqrt(_HS)
_LOG2E = math.log2(math.e)
_LOGZ = -0.5 * math.log(2.0 * math.pi)


def _mlp_t(ws, zt, g1):
    (w1, b1), (w2, b2), (w3, b3) = ws
    t1 = jnp.tanh(jnp.dot(w1, zt, preferred_element_type=jnp.float32) + b1)
    u1 = g1 * (1.0 - t1 * t1)
    t2 = jnp.tanh(jnp.dot(w2, t1, preferred_element_type=jnp.float32) + b2)
    u2 = jnp.dot(w2, u1, preferred_element_type=jnp.float32) * (1.0 - t2 * t2)
    out = jnp.dot(w3, t2, preferred_element_type=jnp.float32) + b3
    dout = jnp.dot(w3, u2, preferred_element_type=jnp.float32)
    return out, dout


def _dotg(a, b, dims):
    return jax.lax.dot_general(a, b, (dims, ((), ())),
                               preferred_element_type=jnp.float32)


def _vf_step(kw, qw, vw, wm, bm, et, g1s, ones_row, zt):
    gk, gq, gv = g1s
    kt, dkt = _mlp_t(kw, zt, gk)
    qt, dqt = _mlp_t(qw, zt, gq)
    vt, dvt = _mlp_t(vw, zt, gv)

    rt_heads = []
    drt_heads = []
    for h in range(_H):
        sl = slice(h * _HS, (h + 1) * _HS)
        qh = qt[sl] * (_INV_SQRT_HS * _LOG2E)
        kh = kt[sl]
        kh4 = kt[sl] * _INV_SQRT_HS
        dkh4 = dkt[sl] * _INV_SQRT_HS
        vh = vt[sl]
        dvh = dvt[sl]

        st = _dotg(kh, qh, ((0,), (0,)))
        m = jnp.max(st, axis=0, keepdims=True)
        p = jnp.exp2(st - m)

        dst = _dotg(jnp.concatenate([kh4, dkh4], axis=0),
                    jnp.concatenate([dqt[sl], qt[sl]], axis=0),
                    ((0,), (0,)))
        u = p * dst

        av3 = _dotg(jnp.concatenate([vh, dvh, ones_row], axis=0), p,
                    ((1,), (0,)))
        rd = 1.0 / av3[2 * _HS:]
        attn_t = av3[:_HS] * rd
        a_dv = av3[_HS:2 * _HS] * rd

        vu2 = _dotg(jnp.concatenate([vh, ones_row], axis=0), u,
                    ((1,), (0,)))
        dav = (vu2[:_HS] - attn_t * vu2[_HS:]) * rd

        rt_heads.append(qt[sl] + attn_t)
        drt_heads.append(dqt[sl] + a_dv + dav)

    rt = jnp.concatenate(rt_heads, axis=0)
    drt = jnp.concatenate(drt_heads, axis=0)
    tt = jnp.tanh(rt)
    dx = jnp.dot(wm, tt, preferred_element_type=jnp.float32) + bm
    ddx = jnp.dot(wm, drt * (1.0 - tt * tt),
                  preferred_element_type=jnp.float32)
    div = jnp.sum(ddx * et)
    return dx, div


def _cnf_kernel(xt_ref, et_ref, *rest):
    out_ref = rest[-1]
    wrefs = rest[:-1]

    zts = [xt_ref[g] for g in range(_GB)]
    lps = [jnp.float32(0.0)] * _GB
    ones_row = jnp.ones((1, _N), dtype=jnp.float32)

    per_blk = 21
    for i in range(_NUM_BLOCKS):
        r = wrefs[i * per_blk:(i + 1) * per_blk]
        kw = [(r[0][...], r[1][...]), (r[2][...], r[3][...]), (r[4][...], r[5][...])]
        qw = [(r[6][...], r[7][...]), (r[8][...], r[9][...]), (r[10][...], r[11][...])]
        vw = [(r[12][...], r[13][...]), (r[14][...], r[15][...]), (r[16][...], r[17][...])]
        wm, bm = r[18][...], r[19][...]
        sqrt_t = r[20][0, 0]
        dt = (sqrt_t * sqrt_t) / (_STEPS - 1)
        ets = [et_ref[i, g] for g in range(_GB)]
        g1s = [(jnp.dot(kw[0][0], ets[g], preferred_element_type=jnp.float32),
                jnp.dot(qw[0][0], ets[g], preferred_element_type=jnp.float32),
                jnp.dot(vw[0][0], ets[g], preferred_element_type=jnp.float32))
               for g in range(_GB)]

        def step(_, carry, kw=kw, qw=qw, vw=vw, wm=wm, bm=bm,
                 ets=ets, g1s=g1s, dt=dt):
            new = []
            for g in range(_GB):
                zt, lp = carry[g]
                dx, div = _vf_step(kw, qw, vw, wm, bm, ets[g], g1s[g],
                                   ones_row, zt)
                new.append((zt + dt * dx, lp - dt * div))
            return tuple(new)

        carry = jax.lax.fori_loop(
            0, _STEPS - 1, step,
            tuple((zts[g], lps[g]) for g in range(_GB)))
        zts = [carry[g][0] for g in range(_GB)]
        lps = [carry[g][1] for g in range(_GB)]

    for g in range(_GB):
        logpz = jnp.sum(_LOGZ - 0.5 * zts[g] * zts[g])
        out_ref[g] = jnp.reshape(logpz - lps[g], (1, 1))


def kernel(x, e, params):
    xt = x.transpose(0, 2, 1)
    et = e.transpose(0, 1, 3, 2)

    flat = []
    for blk in params:
        for name in ("K", "Q", "V"):
            for (w, b) in blk[name]:
                flat.append(w.T)
                flat.append(b.reshape(-1, 1))
        wm, bm = blk["M"][0]
        flat.append(wm.T)
        flat.append(bm.reshape(-1, 1))
        flat.append(blk["sqrt_T"].reshape(1, 1))

    n = _N
    n_prog = _B // _GB
    in_specs = [
        pl.BlockSpec((_GB, _C, n), lambda b: (b, 0, 0)),
        pl.BlockSpec((_NUM_BLOCKS, _GB, _C, n), lambda b: (0, b, 0, 0)),
    ]
    for a in flat:
        in_specs.append(
            pl.BlockSpec(a.shape,
                         functools.partial(lambda nd, b: (0,) * nd, a.ndim)))

    out = pl.pallas_call(
        _cnf_kernel,
        grid=(n_prog,),
        in_specs=in_specs,
        out_specs=pl.BlockSpec((_GB, 1, 1), lambda b: (b, 0, 0)),
        out_shape=jax.ShapeDtypeStruct((_B, 1, 1), jnp.float32),
        compiler_params=pltpu.CompilerParams(
            dimension_semantics=("parallel",),
            vmem_limit_bytes=56 * 1024 * 1024,
        ),
    )(xt, et, *flat)
    return out.reshape(_B)
